# baseline (speedup 1.0000x reference)
import functools

import jax
import jax.numpy as jnp
from jax.experimental import pallas as pl
from jax.experimental.pallas import tpu as pltpu

_NUM_NUMERICAL = 2032
_NEG_SLOPE = 0.01
_SBLK = 1024


def _gate_kernel(num_numerical, pe_ref, codec_ref, te_ref, mask_ref, w1_ref,
                 b1_ref, w2_ref, b2_ref, g_ref, ie_ref,
                 w1b_s, codec_s, te_s, wm_s):
    step = pl.program_id(0)
    sblk, c = pe_ref.shape
    bsz = codec_ref.shape[0]
    e = w2_ref.shape[1]

    bf = jnp.bfloat16
    dn = (((1,), (0,)), ((), ()))

    @pl.when(step == 0)
    def _():
        w1b_s[...] = w1_ref[c:2 * c, :].astype(bf)
        codec_s[...] = jax.lax.dot_general(
            codec_ref[...].astype(bf), w1_ref[0:c, :].astype(bf), dn,
            preferred_element_type=jnp.float32)
        te_s[...] = jax.lax.dot_general(
            te_ref[...].astype(bf), w1_ref[2 * c:3 * c, :].astype(bf), dn,
            preferred_element_type=jnp.float32)
        wm_s[...] = w1_ref[3 * c, :].astype(bf).astype(jnp.float32)[None, :]

    pe_part = jax.lax.dot_general(pe_ref[...].astype(bf), w1b_s[...], dn,
                                  preferred_element_type=jnp.float32)
    codec_part = codec_s[...]
    te_part = te_s[...]
    w_mask = wm_s[0, :]

    pos = step * sblk + jax.lax.broadcasted_iota(jnp.int32, (sblk, 1), 0)
    is_cat = (pos >= num_numerical).astype(jnp.float32)
    te_sel = te_part[0][None, :] + is_cat * (te_part[1] - te_part[0])[None, :]
    base = pe_part + te_sel + b1_ref[...]

    exp_idx = jax.lax.broadcasted_iota(jnp.int32, (e, sblk), 0)
    imin = jnp.int32(-2**31)
    for b in range(bsz):
        mask_b = mask_ref[b, :].astype(jnp.float32)
        pre = base + codec_part[b][None, :] + mask_b[:, None] * w_mask[None, :]
        h = jnp.maximum(pre, _NEG_SLOPE * pre)
        logits = jax.lax.dot_general(w2_ref[...].astype(bf), h.astype(bf),
                                     (((0,), (1,)), ((), ())),
                                     preferred_element_type=jnp.float32)
        logits = (logits + b2_ref[...]) * mask_b[None, :]
        bits = jax.lax.bitcast_convert_type(logits, jnp.int32)
        keys = jnp.where(bits < 0, (~bits) ^ imin, bits)
        k1 = jnp.max(keys, axis=0)
        i1 = jnp.min(jnp.where(keys == k1[None, :], exp_idx, e), axis=0)
        restk = jnp.where(exp_idx == i1[None, :], imin, keys)
        k2 = jnp.max(restk, axis=0)
        i2 = jnp.min(jnp.where(restk == k2[None, :], exp_idx, e), axis=0)
        m1 = jax.lax.bitcast_convert_type(
            jnp.where(k1 < 0, ~(k1 ^ imin), k1), jnp.float32)
        m2 = jax.lax.bitcast_convert_type(
            jnp.where(k2 < 0, ~(k2 ^ imin), k2), jnp.float32)
        ex = jnp.exp(m2 - m1)
        denom = 1.0 + ex
        g_ref[b, :, 0] = 1.0 / denom
        g_ref[b, :, 1] = ex / denom
        ie_ref[b, :, 0] = i1
        ie_ref[b, :, 1] = i2


def kernel(codec, mask_pos, pos_emb, type_emb, W1, b1, W2, b2):
    bsz, s = mask_pos.shape
    c = codec.shape[1]
    h = W1.shape[1]
    e = W2.shape[1]
    num_numerical = _NUM_NUMERICAL

    b1_2d = b1.reshape(1, h)
    b2_2d = b2.reshape(e, 1)

    grid = s // _SBLK
    out_shapes = (
        jax.ShapeDtypeStruct((bsz, s, 2), jnp.float32),
        jax.ShapeDtypeStruct((bsz, s, 2), jnp.int32),
    )
    out_spec = pl.BlockSpec((bsz, _SBLK, 2), lambda i: (0, i, 0))
    gates, experts = pl.pallas_call(
        functools.partial(_gate_kernel, num_numerical),
        grid=(grid,),
        in_specs=[
            pl.BlockSpec((_SBLK, c), lambda i: (i, 0)),
            pl.BlockSpec((bsz, c), lambda i: (0, 0)),
            pl.BlockSpec((2, c), lambda i: (0, 0)),
            pl.BlockSpec((bsz, _SBLK), lambda i: (0, i)),
            pl.BlockSpec(W1.shape, lambda i: (0, 0)),
            pl.BlockSpec((1, h), lambda i: (0, 0)),
            pl.BlockSpec((h, e), lambda i: (0, 0)),
            pl.BlockSpec((e, 1), lambda i: (0, 0)),
        ],
        out_specs=(out_spec, out_spec),
        out_shape=out_shapes,
        scratch_shapes=[
            pltpu.VMEM((c, h), jnp.bfloat16),
            pltpu.VMEM((bsz, h), jnp.float32),
            pltpu.VMEM((2, h), jnp.float32),
            pltpu.VMEM((1, h), jnp.float32),
        ],
    )(pos_emb, codec, type_emb, mask_pos, W1, b1_2d, W2, b2_2d)

    return gates, experts, mask_pos


# speedup vs baseline: 1.3816x; 1.3816x over previous
import functools

import jax
import jax.numpy as jnp
from jax.experimental import pallas as pl
from jax.experimental.pallas import tpu as pltpu

_NUM_NUMERICAL = 2032
_NEG_SLOPE = 0.01
_SBLK = 1024


def _gate_kernel(num_numerical, pe_ref, codec_ref, te_ref, mask_ref, w1_ref,
                 b1_ref, w2_ref, b2_ref, g1_ref, g2_ref, i1_ref, i2_ref,
                 w1b_s, codec_s, te_s, wm_s):
    step = pl.program_id(0)
    sblk, c = pe_ref.shape
    bsz = codec_ref.shape[0]
    e = w2_ref.shape[1]

    bf = jnp.bfloat16
    dn = (((1,), (0,)), ((), ()))

    @pl.when(step == 0)
    def _():
        w1b_s[...] = w1_ref[c:2 * c, :].astype(bf)
        codec_s[...] = jax.lax.dot_general(
            codec_ref[...].astype(bf), w1_ref[0:c, :].astype(bf), dn,
            preferred_element_type=jnp.float32)
        te_s[...] = jax.lax.dot_general(
            te_ref[...].astype(bf), w1_ref[2 * c:3 * c, :].astype(bf), dn,
            preferred_element_type=jnp.float32)
        wm_s[...] = w1_ref[3 * c, :].astype(bf).astype(jnp.float32)[None, :]

    pe_part = jax.lax.dot_general(pe_ref[...].astype(bf), w1b_s[...], dn,
                                  preferred_element_type=jnp.float32)
    codec_part = codec_s[...]
    te_part = te_s[...]
    w_mask = wm_s[0, :]

    pos = step * sblk + jax.lax.broadcasted_iota(jnp.int32, (sblk, 1), 0)
    is_cat = (pos >= num_numerical).astype(jnp.float32)
    te_sel = te_part[0][None, :] + is_cat * (te_part[1] - te_part[0])[None, :]
    base = pe_part + te_sel + b1_ref[...]

    exp_idx = jax.lax.broadcasted_iota(jnp.int32, (e, sblk), 0)
    imin = jnp.int32(-2**31)
    for b in range(bsz):
        mask_b = mask_ref[b, :].astype(jnp.float32)
        pre = base + codec_part[b][None, :] + mask_b[:, None] * w_mask[None, :]
        h = jnp.maximum(pre, _NEG_SLOPE * pre)
        logits = jax.lax.dot_general(w2_ref[...].astype(bf), h.astype(bf),
                                     (((0,), (1,)), ((), ())),
                                     preferred_element_type=jnp.float32)
        logits = (logits + b2_ref[...]) * mask_b[None, :]
        bits = jax.lax.bitcast_convert_type(logits, jnp.int32)
        keys = jnp.where(bits < 0, (~bits) ^ imin, bits)
        k1 = jnp.max(keys, axis=0)
        i1 = jnp.min(jnp.where(keys == k1[None, :], exp_idx, e), axis=0)
        restk = jnp.where(exp_idx == i1[None, :], imin, keys)
        k2 = jnp.max(restk, axis=0)
        i2 = jnp.min(jnp.where(restk == k2[None, :], exp_idx, e), axis=0)
        m1 = jax.lax.bitcast_convert_type(
            jnp.where(k1 < 0, ~(k1 ^ imin), k1), jnp.float32)
        m2 = jax.lax.bitcast_convert_type(
            jnp.where(k2 < 0, ~(k2 ^ imin), k2), jnp.float32)
        ex = jnp.exp(m2 - m1)
        denom = 1.0 + ex
        g1_ref[b, :] = 1.0 / denom
        g2_ref[b, :] = ex / denom
        i1_ref[b, :] = i1
        i2_ref[b, :] = i2


def kernel(codec, mask_pos, pos_emb, type_emb, W1, b1, W2, b2):
    bsz, s = mask_pos.shape
    c = codec.shape[1]
    h = W1.shape[1]
    e = W2.shape[1]
    num_numerical = _NUM_NUMERICAL

    b1_2d = b1.reshape(1, h)
    b2_2d = b2.reshape(e, 1)

    grid = s // _SBLK
    out_shapes = (
        jax.ShapeDtypeStruct((bsz, s), jnp.float32),
        jax.ShapeDtypeStruct((bsz, s), jnp.float32),
        jax.ShapeDtypeStruct((bsz, s), jnp.int32),
        jax.ShapeDtypeStruct((bsz, s), jnp.int32),
    )
    out_spec = pl.BlockSpec((bsz, _SBLK), lambda i: (0, i))
    g1, g2, i1, i2 = pl.pallas_call(
        functools.partial(_gate_kernel, num_numerical),
        grid=(grid,),
        in_specs=[
            pl.BlockSpec((_SBLK, c), lambda i: (i, 0)),
            pl.BlockSpec((bsz, c), lambda i: (0, 0)),
            pl.BlockSpec((2, c), lambda i: (0, 0)),
            pl.BlockSpec((bsz, _SBLK), lambda i: (0, i)),
            pl.BlockSpec(W1.shape, lambda i: (0, 0)),
            pl.BlockSpec((1, h), lambda i: (0, 0)),
            pl.BlockSpec((h, e), lambda i: (0, 0)),
            pl.BlockSpec((e, 1), lambda i: (0, 0)),
        ],
        out_specs=(out_spec, out_spec, out_spec, out_spec),
        out_shape=out_shapes,
        scratch_shapes=[
            pltpu.VMEM((c, h), jnp.bfloat16),
            pltpu.VMEM((bsz, h), jnp.float32),
            pltpu.VMEM((2, h), jnp.float32),
            pltpu.VMEM((1, h), jnp.float32),
        ],
    )(pos_emb, codec, type_emb, mask_pos, W1, b1_2d, W2, b2_2d)

    gates = jnp.stack([g1, g2], axis=-1)
    experts = jnp.stack([i1, i2], axis=-1)
    return gates, experts, mask_pos
